# Initial kernel scaffold; baseline (speedup 1.0000x reference)
import jax
import jax.numpy as jnp
from jax.experimental import pallas as pl


def kernel(relation_triplets, rel_emb, proj1_W, proj1_b, l0_attn_W, l0_attn_b, l0_attn_bin, l0_attn_vec, l0_aggr_W, l0_aggr_b, l0_res_W, l0_res_b, l1_attn_W, l1_attn_b, l1_attn_bin, l1_attn_vec, l1_aggr_W, l1_aggr_b, l1_res_W, l1_res_b):
    raise NotImplementedError("write your pallas kernel here")


# speedup vs baseline: 5.6878x; 5.6878x over previous
import functools

import jax
import jax.numpy as jnp
from jax import lax
from jax.experimental import pallas as pl
from jax.experimental.pallas import tpu as pltpu
from jax.experimental.pallas import tpu_sc as plsc

NR = 10000
DIM = 256
H = 8
DH = 32
NP = 10240
L = 16
NC = 2
NS = 16
E_PAD = 163840
CH = 128
EPT1 = E_PAD // (NC * NS)
NCH1 = EPT1 // CH
EPT2 = E_PAD // NS
NCH2 = EPT2 // CH
ACC_W = 144
E_REAL = 160000
RPT = NR // NS
WCH = 125

_mesh = plsc.VectorSubcoreMesh(
    core_axis_name="c", subcore_axis_name="s", num_cores=NC, num_subcores=NS
)
_sc_params = pltpu.CompilerParams(
    use_tc_tiling_on_sc=False, needs_layout_passes=False
)


def _leaky(x):
    return jnp.where(x >= 0, x, 0.2 * x)


def _proj_body(e_ref, w_ref, b_ref, o_ref):
    o_ref[...] = jax.nn.relu(
        jnp.dot(e_ref[...], w_ref[...], preferred_element_type=jnp.float32)
        + b_ref[...]
    )


def _tc_proj(embp, W, b):
    blk = 256
    return pl.pallas_call(
        _proj_body,
        grid=(NP // blk,),
        in_specs=[
            pl.BlockSpec((blk, DIM), lambda i: (i, 0)),
            pl.BlockSpec((DIM, DIM), lambda i: (0, 0)),
            pl.BlockSpec((1, DIM), lambda i: (0, 0)),
        ],
        out_specs=pl.BlockSpec((blk, DIM), lambda i: (i, 0)),
        out_shape=jax.ShapeDtypeStruct((NP, DIM), jnp.float32),
    )(embp, W, b)


def _mm_body(e_ref, wh_ref, wt_ref, ab_ref, aw_ref, agb_ref, rw_ref, rb_ref,
             a_ref, b_ref, c0_ref, c1_ref, r_ref):
    e = e_ref[...]
    a_ref[...] = jnp.dot(e, wh_ref[...], preferred_element_type=jnp.float32)
    b_ref[...] = (
        jnp.dot(e, wt_ref[...], preferred_element_type=jnp.float32) + ab_ref[...]
    )
    c = jnp.dot(e, aw_ref[...], preferred_element_type=jnp.float32) + agb_ref[...]
    c0_ref[...] = c[:, :128]
    c1_ref[...] = c[:, 128:]
    r_ref[...] = jax.nn.relu(
        jnp.dot(e, rw_ref[...], preferred_element_type=jnp.float32) + rb_ref[...]
    )


def _tc_mm(embp, Wh, Wt, attn_b, aggr_W, aggr_b, res_W, res_b):
    blk = 256
    wspec = pl.BlockSpec((DIM, DIM), lambda i: (0, 0))
    bspec = pl.BlockSpec((1, DIM), lambda i: (0, 0))
    ospec = pl.BlockSpec((blk, DIM), lambda i: (i, 0))
    hspec = pl.BlockSpec((blk, 128), lambda i: (i, 0))
    return pl.pallas_call(
        _mm_body,
        grid=(NP // blk,),
        in_specs=[
            pl.BlockSpec((blk, DIM), lambda i: (i, 0)),
            wspec, wspec, bspec, wspec, bspec, wspec, bspec,
        ],
        out_specs=[ospec, ospec, hspec, hspec, ospec],
        out_shape=[
            jax.ShapeDtypeStruct((NP, DIM), jnp.float32),
            jax.ShapeDtypeStruct((NP, DIM), jnp.float32),
            jax.ShapeDtypeStruct((NP, 128), jnp.float32),
            jax.ShapeDtypeStruct((NP, 128), jnp.float32),
            jax.ShapeDtypeStruct((NP, DIM), jnp.float32),
        ],
    )(embp, Wh, Wt, attn_b, aggr_W, aggr_b, res_W, res_b)


def _combine_body(m0_ref, s0_ref, m1_ref, s1_ref, r_ref, o_ref):
    eps = 1e-16
    outs = []
    for m_ref, s_ref in ((m0_ref, s0_ref), (m1_ref, s1_ref)):
        m = m_ref[...]
        s = s_ref[...]
        d = jnp.concatenate(
            [jnp.broadcast_to(s[:, k:k + 1], (m.shape[0], DH)) for k in range(4)],
            axis=1,
        )
        outs.append(m / (d + eps))
    o = jnp.concatenate(outs, axis=1)
    o_ref[...] = jax.nn.relu(o) + r_ref[...]


def _tc_combine(m0, s0, m1, s1, R):
    blk = 80
    mspec = pl.BlockSpec((blk, 128), lambda i: (i, 0))
    sspec = pl.BlockSpec((blk, 4), lambda i: (i, 0))
    return pl.pallas_call(
        _combine_body,
        grid=(NR // blk,),
        in_specs=[mspec, sspec, mspec, sspec,
                  pl.BlockSpec((blk, DIM), lambda i: (i, 0))],
        out_specs=pl.BlockSpec((blk, DIM), lambda i: (i, 0)),
        out_shape=jax.ShapeDtypeStruct((NR, DIM), jnp.float32),
    )(m0, s0, m1, s1, R)


def _sc_logits_body(hp, tp, bp, A, B, binp, vecb,
                    logitsT, tilemax,
                    hbuf, tbuf, bbuf, arows, brows, binrows, vecv, lbuf,
                    maxb, sem):
    cid = lax.axis_index("c")
    sid = lax.axis_index("s")
    wid = sid * NC + cid
    base = wid * EPT1
    iot = lax.iota(jnp.int32, L)
    pltpu.sync_copy(vecb, vecv)

    def chunk_body(ci, maxc):
        off = base + ci * CH
        pltpu.sync_copy(hp.at[pl.ds(off, CH)], hbuf)
        pltpu.sync_copy(tp.at[pl.ds(off, CH)], tbuf)
        pltpu.sync_copy(bp.at[pl.ds(off, CH)], bbuf)
        pltpu.async_copy(A.at[hbuf], arows, sem).wait()
        pltpu.async_copy(B.at[tbuf], brows, sem).wait()
        pltpu.async_copy(binp.at[bbuf], binrows, sem).wait()

        def group_body(g, mc):
            eidx = g * L + iot
            new = []
            for k in range(H):
                acc = jnp.zeros((L,), jnp.float32)
                for d2 in range(DH):
                    d = k * DH + d2
                    cold = jnp.full((L,), d, jnp.int32)
                    a = plsc.load_gather(arows, [eidx, cold])
                    b = plsc.load_gather(brows, [eidx, cold])
                    acc = acc + _leaky(a + b) * vecv[d]
                bv = plsc.load_gather(
                    binrows, [eidx, jnp.full((L,), k, jnp.int32)]
                )
                lg = acc + _leaky(bv)
                lbuf[k, pl.ds(g * L, L)] = lg
                new.append(jnp.maximum(mc[k], lg))
            return tuple(new)

        maxc = pl.loop(0, CH // L, init_carry=maxc)(group_body)
        pltpu.sync_copy(lbuf, logitsT.at[:, pl.ds(off, CH)])
        return maxc

    init = tuple(jnp.full((L,), -1e30, jnp.float32) for _ in range(H))
    maxc = pl.loop(0, NCH1, init_carry=init)(chunk_body)
    for k in range(H):
        maxb[k, :] = maxc[k]
    pltpu.sync_copy(maxb, tilemax.at[wid])


def _sc_logits(hp, tp, bp, A, B, binp, vecb):
    f = functools.partial(
        pl.kernel,
        _sc_logits_body,
        out_type=[
            jax.ShapeDtypeStruct((H, E_PAD), jnp.float32),
            jax.ShapeDtypeStruct((NC * NS, H, L), jnp.float32),
        ],
        mesh=_mesh,
        scratch_types=[
            pltpu.VMEM((CH,), jnp.int32),
            pltpu.VMEM((CH,), jnp.int32),
            pltpu.VMEM((CH,), jnp.int32),
            pltpu.VMEM((CH, DIM), jnp.float32),
            pltpu.VMEM((CH, DIM), jnp.float32),
            pltpu.VMEM((CH, L), jnp.float32),
            pltpu.VMEM((DIM, L), jnp.float32),
            pltpu.VMEM((H, CH), jnp.float32),
            pltpu.VMEM((H, L), jnp.float32),
            pltpu.SemaphoreType.DMA,
        ],
        compiler_params=_sc_params,
    )()
    return f(hp, tp, bp, A, B, binp, vecb)


def _sc_aggr_body(hp, tp, logitsT, tilemax, C0, C1,
                  accum,
                  hbuf, tbuf, lbuf, crows, msg, tmaxv, oshared, sem):
    cid = lax.axis_index("c")
    sid = lax.axis_index("s")
    iot = lax.iota(jnp.int32, L)
    pltpu.sync_copy(tilemax, tmaxv)

    Ks = []
    for k in range(4):
        def wbody(w, acc, _k=k):
            return jnp.maximum(acc, tmaxv[w, cid * 4 + _k, :])
        acc = pl.loop(0, NC * NS,
                      init_carry=jnp.full((L,), -1e30, jnp.float32))(wbody)
        Ks.append(jnp.max(acc))

    @pl.loop(0, CH)
    def _(r):
        for j in range(ACC_W // L):
            msg[r, pl.ds(j * L, L)] = jnp.zeros((L,), jnp.float32)

    for q in range(RPT // WCH):
        pltpu.sync_copy(
            msg.at[pl.ds(0, WCH)], oshared.at[pl.ds(sid * RPT + q * WCH, WCH)]
        )
    plsc.subcore_barrier()

    base = sid * EPT2

    @pl.loop(0, NCH2)
    def chunk_body(ci):
        off = base + ci * CH
        pltpu.sync_copy(hp.at[pl.ds(off, CH)], hbuf)
        pltpu.sync_copy(tp.at[pl.ds(off, CH)], tbuf)
        pltpu.sync_copy(logitsT.at[pl.ds(cid * 4, 4), pl.ds(off, CH)], lbuf)

        @pl.when(cid == 0)
        def _():
            pltpu.async_copy(C0.at[tbuf], crows, sem).wait()

        @pl.when(cid == 1)
        def _():
            pltpu.async_copy(C1.at[tbuf], crows, sem).wait()

        @pl.loop(0, CH // L)
        def group_body(g):
            eidx = g * L + iot
            live = (off + eidx) < E_REAL
            for k in range(4):
                lg = lbuf[k, pl.ds(g * L, L)]
                val = jnp.where(live, jnp.exp(lg - Ks[k]), 0.0)
                plsc.store_scatter(
                    msg, [eidx, jnp.full((L,), 128 + k, jnp.int32)], val
                )
                for d2 in range(DH):
                    d = k * DH + d2
                    cold = jnp.full((L,), d, jnp.int32)
                    cv = plsc.load_gather(crows, [eidx, cold])
                    plsc.store_scatter(msg, [eidx, cold], cv * val)

        pltpu.sync_copy(msg, oshared.at[hbuf], add=True)

    plsc.subcore_barrier()
    for q in range(RPT // WCH):
        rows = sid * RPT + q * WCH
        pltpu.sync_copy(oshared.at[pl.ds(rows, WCH)], msg.at[pl.ds(0, WCH)])
        pltpu.sync_copy(msg.at[pl.ds(0, WCH)], accum.at[cid, pl.ds(rows, WCH)])


def _sc_aggregate(hp, tp, logitsT, tilemax, C0, C1):
    f = functools.partial(
        pl.kernel,
        _sc_aggr_body,
        out_type=jax.ShapeDtypeStruct((NC, NR, ACC_W), jnp.float32),
        mesh=_mesh,
        scratch_types=[
            pltpu.VMEM((CH,), jnp.int32),
            pltpu.VMEM((CH,), jnp.int32),
            pltpu.VMEM((4, CH), jnp.float32),
            pltpu.VMEM((CH, 128), jnp.float32),
            pltpu.VMEM((CH, ACC_W), jnp.float32),
            pltpu.VMEM((NC * NS, H, L), jnp.float32),
            pltpu.VMEM_SHARED((NR, ACC_W), jnp.float32),
            pltpu.SemaphoreType.DMA,
        ],
        compiler_params=_sc_params,
    )()
    return f(hp, tp, logitsT, tilemax, C0, C1)


def kernel(relation_triplets, rel_emb, proj1_W, proj1_b,
           l0_attn_W, l0_attn_b, l0_attn_bin, l0_attn_vec,
           l0_aggr_W, l0_aggr_b, l0_res_W, l0_res_b,
           l1_attn_W, l1_attn_b, l1_attn_bin, l1_attn_vec,
           l1_aggr_W, l1_aggr_b, l1_res_W, l1_res_b):
    E = relation_triplets.shape[0]
    tri = relation_triplets.astype(jnp.int32)
    pad = E_PAD - E
    hp = jnp.concatenate([tri[:, 0], jnp.zeros((pad,), jnp.int32)])
    tp = jnp.concatenate([tri[:, 1], jnp.zeros((pad,), jnp.int32)])
    bp = jnp.concatenate([tri[:, 2], jnp.zeros((pad,), jnp.int32)])

    embp = jnp.pad(rel_emb, ((0, NP - NR), (0, 0)))
    emb = _tc_proj(embp, proj1_W, proj1_b.reshape(1, DIM))

    layers = [
        (l0_attn_W, l0_attn_b, l0_attn_bin, l0_attn_vec,
         l0_aggr_W, l0_aggr_b, l0_res_W, l0_res_b),
        (l1_attn_W, l1_attn_b, l1_attn_bin, l1_attn_vec,
         l1_aggr_W, l1_aggr_b, l1_res_W, l1_res_b),
    ]
    out = None
    for (attn_W, attn_b, attn_bin, attn_vec,
         aggr_W, aggr_b, res_W, res_b) in layers:
        A, Bm, C0, C1, R = _tc_mm(
            emb, attn_W[:DIM], attn_W[DIM:], attn_b.reshape(1, DIM),
            aggr_W, aggr_b.reshape(1, DIM), res_W, res_b.reshape(1, DIM),
        )
        nbin = attn_bin.shape[0]
        binp = jnp.pad(attn_bin.reshape(nbin, H), ((0, 0), (0, L - H)))
        vecb = jnp.broadcast_to(attn_vec.reshape(DIM, 1), (DIM, L))
        logitsT, tilemax = _sc_logits(hp, tp, bp, A, Bm, binp, vecb)
        accum = _sc_aggregate(hp, tp, logitsT, tilemax, C0, C1)
        out = _tc_combine(
            accum[0, :, :128], accum[0, :, 128:132],
            accum[1, :, :128], accum[1, :, 128:132], R,
        )
        emb = jnp.pad(out, ((0, NP - NR), (0, 0)))
    return out


# speedup vs baseline: 7.0580x; 1.2409x over previous
import functools

import jax
import jax.numpy as jnp
from jax import lax
from jax.experimental import pallas as pl
from jax.experimental.pallas import tpu as pltpu
from jax.experimental.pallas import tpu_sc as plsc

NR = 10000
DIM = 256
H = 8
DH = 32
NP = 10240
L = 16
NC = 2
NS = 16
E_PAD = 163840
CH1 = 80
CH = 64
EPT1 = E_PAD // (NC * NS)
NCH1 = EPT1 // CH1
EPT2 = E_PAD // NS
NCH2 = EPT2 // CH
ACC_W = 144
E_REAL = 160000
RPT = NR // NS
WCH = 25

_mesh = plsc.VectorSubcoreMesh(
    core_axis_name="c", subcore_axis_name="s", num_cores=NC, num_subcores=NS
)
_sc_params = pltpu.CompilerParams(
    use_tc_tiling_on_sc=False, needs_layout_passes=False
)


def _leaky(x):
    return jnp.where(x >= 0, x, 0.2 * x)


def _proj_body(e_ref, w_ref, b_ref, o_ref):
    o_ref[...] = jax.nn.relu(
        jnp.dot(e_ref[...], w_ref[...], preferred_element_type=jnp.float32)
        + b_ref[...]
    )


def _tc_proj(embp, W, b):
    blk = 256
    return pl.pallas_call(
        _proj_body,
        grid=(NP // blk,),
        in_specs=[
            pl.BlockSpec((blk, DIM), lambda i: (i, 0)),
            pl.BlockSpec((DIM, DIM), lambda i: (0, 0)),
            pl.BlockSpec((1, DIM), lambda i: (0, 0)),
        ],
        out_specs=pl.BlockSpec((blk, DIM), lambda i: (i, 0)),
        out_shape=jax.ShapeDtypeStruct((NP, DIM), jnp.float32),
    )(embp, W, b)


def _mm_body(e_ref, wh_ref, wt_ref, ab_ref, aw_ref, agb_ref, rw_ref, rb_ref,
             a_ref, b_ref, c0_ref, c1_ref, r_ref):
    e = e_ref[...]
    a_ref[...] = jnp.dot(e, wh_ref[...], preferred_element_type=jnp.float32)
    b_ref[...] = (
        jnp.dot(e, wt_ref[...], preferred_element_type=jnp.float32) + ab_ref[...]
    )
    c = jnp.dot(e, aw_ref[...], preferred_element_type=jnp.float32) + agb_ref[...]
    c0_ref[...] = c[:, :128]
    c1_ref[...] = c[:, 128:]
    r_ref[...] = jax.nn.relu(
        jnp.dot(e, rw_ref[...], preferred_element_type=jnp.float32) + rb_ref[...]
    )


def _tc_mm(embp, Wh, Wt, attn_b, aggr_W, aggr_b, res_W, res_b):
    blk = 256
    wspec = pl.BlockSpec((DIM, DIM), lambda i: (0, 0))
    bspec = pl.BlockSpec((1, DIM), lambda i: (0, 0))
    ospec = pl.BlockSpec((blk, DIM), lambda i: (i, 0))
    hspec = pl.BlockSpec((blk, 128), lambda i: (i, 0))
    return pl.pallas_call(
        _mm_body,
        grid=(NP // blk,),
        in_specs=[
            pl.BlockSpec((blk, DIM), lambda i: (i, 0)),
            wspec, wspec, bspec, wspec, bspec, wspec, bspec,
        ],
        out_specs=[ospec, ospec, hspec, hspec, ospec],
        out_shape=[
            jax.ShapeDtypeStruct((NP, DIM), jnp.float32),
            jax.ShapeDtypeStruct((NP, DIM), jnp.float32),
            jax.ShapeDtypeStruct((NP, 128), jnp.float32),
            jax.ShapeDtypeStruct((NP, 128), jnp.float32),
            jax.ShapeDtypeStruct((NP, DIM), jnp.float32),
        ],
    )(embp, Wh, Wt, attn_b, aggr_W, aggr_b, res_W, res_b)


def _combine_body(m0_ref, s0_ref, m1_ref, s1_ref, r_ref, o_ref):
    eps = 1e-16
    outs = []
    for m_ref, s_ref in ((m0_ref, s0_ref), (m1_ref, s1_ref)):
        m = m_ref[...]
        s = s_ref[...]
        d = jnp.concatenate(
            [jnp.broadcast_to(s[:, k:k + 1], (m.shape[0], DH)) for k in range(4)],
            axis=1,
        )
        outs.append(m / (d + eps))
    o = jnp.concatenate(outs, axis=1)
    o_ref[...] = jax.nn.relu(o) + r_ref[...]


def _tc_combine(m0, s0, m1, s1, R):
    blk = 80
    mspec = pl.BlockSpec((blk, 128), lambda i: (i, 0))
    sspec = pl.BlockSpec((blk, 4), lambda i: (i, 0))
    return pl.pallas_call(
        _combine_body,
        grid=(NR // blk,),
        in_specs=[mspec, sspec, mspec, sspec,
                  pl.BlockSpec((blk, DIM), lambda i: (i, 0))],
        out_specs=pl.BlockSpec((blk, DIM), lambda i: (i, 0)),
        out_shape=jax.ShapeDtypeStruct((NR, DIM), jnp.float32),
    )(m0, s0, m1, s1, R)


def _sc_logits_body(hp, tp, bp, A, B, binp, vecb,
                    logitsT, tilemax,
                    hbufs, tbufs, bbufs, arowss, browss, binrowss, vecv,
                    lbuf, maxb, sems):
    cid = lax.axis_index("c")
    sid = lax.axis_index("s")
    wid = sid * NC + cid
    base = wid * EPT1
    iot = lax.iota(jnp.int32, L)
    pltpu.sync_copy(vecb, vecv)

    def issue(off, p):
        pltpu.sync_copy(hp.at[pl.ds(off, CH1)], hbufs[p])
        pltpu.sync_copy(tp.at[pl.ds(off, CH1)], tbufs[p])
        pltpu.sync_copy(bp.at[pl.ds(off, CH1)], bbufs[p])
        pltpu.async_copy(A.at[hbufs[p]], arowss[p], sems[p])
        pltpu.async_copy(B.at[tbufs[p]], browss[p], sems[p])
        pltpu.async_copy(binp.at[bbufs[p]], binrowss[p], sems[p])

    def drain(p):
        pltpu.make_async_copy(A.at[hbufs[p]], arowss[p], sems[p]).wait()
        pltpu.make_async_copy(B.at[tbufs[p]], browss[p], sems[p]).wait()
        pltpu.make_async_copy(binp.at[bbufs[p]], binrowss[p], sems[p]).wait()

    def compute(off, p, maxc):
        arows, brows, binrows = arowss[p], browss[p], binrowss[p]

        def group_body(g, mc):
            eidx = g * L + iot
            new = []
            for k in range(H):
                acc = jnp.zeros((L,), jnp.float32)
                for d2 in range(DH):
                    d = k * DH + d2
                    cold = jnp.full((L,), d, jnp.int32)
                    a = plsc.load_gather(arows, [eidx, cold])
                    b = plsc.load_gather(brows, [eidx, cold])
                    acc = acc + _leaky(a + b) * vecv[d]
                bv = plsc.load_gather(
                    binrows, [eidx, jnp.full((L,), k, jnp.int32)]
                )
                lg = acc + _leaky(bv)
                lbuf[k, pl.ds(g * L, L)] = lg
                new.append(jnp.maximum(mc[k], lg))
            return tuple(new)

        maxc = pl.loop(0, CH1 // L, init_carry=maxc)(group_body)
        pltpu.sync_copy(lbuf, logitsT.at[:, pl.ds(off, CH1)])
        return maxc

    issue(base, 0)

    def pair_body(cj, maxc):
        for p in range(2):
            ci = 2 * cj + p

            @pl.when(ci + 1 < NCH1)
            def _():
                issue(base + (ci + 1) * CH1, 1 - p)

            drain(p)
            maxc = compute(base + ci * CH1, p, maxc)
        return maxc

    init = tuple(jnp.full((L,), -1e30, jnp.float32) for _ in range(H))
    maxc = pl.loop(0, NCH1 // 2, init_carry=init)(pair_body)
    for k in range(H):
        maxb[k, :] = maxc[k]
    pltpu.sync_copy(maxb, tilemax.at[wid])


def _sc_logits(hp, tp, bp, A, B, binp, vecb):
    ibuf = pltpu.VMEM((CH1,), jnp.int32)
    rbuf = pltpu.VMEM((CH1, DIM), jnp.float32)
    nbuf = pltpu.VMEM((CH1, L), jnp.float32)
    f = functools.partial(
        pl.kernel,
        _sc_logits_body,
        out_type=[
            jax.ShapeDtypeStruct((H, E_PAD), jnp.float32),
            jax.ShapeDtypeStruct((NC * NS, H, L), jnp.float32),
        ],
        mesh=_mesh,
        scratch_types=[
            (ibuf, ibuf), (ibuf, ibuf), (ibuf, ibuf),
            (rbuf, rbuf), (rbuf, rbuf), (nbuf, nbuf),
            pltpu.VMEM((DIM, L), jnp.float32),
            pltpu.VMEM((H, CH1), jnp.float32),
            pltpu.VMEM((H, L), jnp.float32),
            (pltpu.SemaphoreType.DMA, pltpu.SemaphoreType.DMA),
        ],
        compiler_params=_sc_params,
    )()
    return f(hp, tp, bp, A, B, binp, vecb)


def _sc_aggr_body(hp, tp, logitsT, tilemax, C0, C1,
                  accum,
                  hbufs, tbufs, abufs, lbufs, crowss, msgs, tmaxv, oshared,
                  gsems, asems):
    cid = lax.axis_index("c")
    sid = lax.axis_index("s")
    iot = lax.iota(jnp.int32, L)
    pltpu.sync_copy(tilemax, tmaxv)

    Ks = []
    for k in range(4):
        def wbody(w, acc, _k=k):
            return jnp.maximum(acc, tmaxv[w, cid * 4 + _k, :])
        acc = pl.loop(0, NC * NS,
                      init_carry=jnp.full((L,), -1e30, jnp.float32))(wbody)
        Ks.append(jnp.max(acc))

    for p in range(2):
        @pl.loop(0, CH)
        def _(r, _p=p):
            for j in range(ACC_W // L):
                msgs[_p][r, pl.ds(j * L, L)] = jnp.zeros((L,), jnp.float32)

    for q in range(RPT // WCH):
        pltpu.sync_copy(
            msgs[0].at[pl.ds(0, WCH)],
            oshared.at[pl.ds(sid * RPT + q * WCH, WCH)],
        )
    plsc.subcore_barrier()

    base = sid * EPT2

    def issue(off, p):
        pltpu.sync_copy(hp.at[pl.ds(off, CH)], hbufs[p])
        pltpu.sync_copy(tp.at[pl.ds(off, CH)], tbufs[p])
        pltpu.sync_copy(
            logitsT.at[pl.ds(cid * 4, 4), pl.ds(off, CH)], lbufs[p]
        )

        @pl.when(cid == 0)
        def _():
            pltpu.async_copy(C0.at[tbufs[p]], crowss[p], gsems[p])

        @pl.when(cid == 1)
        def _():
            pltpu.async_copy(C1.at[tbufs[p]], crowss[p], gsems[p])

    def drain_gather(p):
        @pl.when(cid == 0)
        def _():
            pltpu.make_async_copy(C0.at[tbufs[p]], crowss[p], gsems[p]).wait()

        @pl.when(cid == 1)
        def _():
            pltpu.make_async_copy(C1.at[tbufs[p]], crowss[p], gsems[p]).wait()

    def drain_add(p):
        pltpu.make_async_copy(
            msgs[p], oshared.at[abufs[p]], asems[p]
        ).wait()

    issue(base, 0)

    def pair_body(cj):
        for p in range(2):
            ci = 2 * cj + p
            off = base + ci * CH

            @pl.when(ci + 1 < NCH2)
            def _():
                issue(base + (ci + 1) * CH, 1 - p)

            drain_gather(p)

            @pl.when(cj > 0)
            def _():
                drain_add(p)

            lbuf, crows, msg = lbufs[p], crowss[p], msgs[p]

            @pl.loop(0, CH // L)
            def group_body(g):
                eidx = g * L + iot
                live = (off + eidx) < E_REAL
                for k in range(4):
                    lg = lbuf[k, pl.ds(g * L, L)]
                    val = jnp.where(live, jnp.exp(lg - Ks[k]), 0.0)
                    plsc.store_scatter(
                        msg, [eidx, jnp.full((L,), 128 + k, jnp.int32)], val
                    )
                    for d2 in range(DH):
                        d = k * DH + d2
                        cold = jnp.full((L,), d, jnp.int32)
                        cv = plsc.load_gather(crows, [eidx, cold])
                        plsc.store_scatter(msg, [eidx, cold], cv * val)

            for j in range(CH // L):
                abufs[p][pl.ds(j * L, L)] = hbufs[p][pl.ds(j * L, L)]
            pltpu.async_copy(msg, oshared.at[abufs[p]], asems[p], add=True)

    pl.loop(0, NCH2 // 2)(pair_body)
    drain_add(0)
    drain_add(1)

    plsc.subcore_barrier()
    for q in range(RPT // WCH):
        rows = sid * RPT + q * WCH
        pltpu.sync_copy(
            oshared.at[pl.ds(rows, WCH)], msgs[0].at[pl.ds(0, WCH)]
        )
        pltpu.sync_copy(
            msgs[0].at[pl.ds(0, WCH)], accum.at[cid, pl.ds(rows, WCH)]
        )


def _sc_aggregate(hp, tp, logitsT, tilemax, C0, C1):
    ibuf = pltpu.VMEM((CH,), jnp.int32)
    f = functools.partial(
        pl.kernel,
        _sc_aggr_body,
        out_type=jax.ShapeDtypeStruct((NC, NR, ACC_W), jnp.float32),
        mesh=_mesh,
        scratch_types=[
            (ibuf, ibuf), (ibuf, ibuf), (ibuf, ibuf),
            (pltpu.VMEM((4, CH), jnp.float32),
             pltpu.VMEM((4, CH), jnp.float32)),
            (pltpu.VMEM((CH, 128), jnp.float32),
             pltpu.VMEM((CH, 128), jnp.float32)),
            (pltpu.VMEM((CH, ACC_W), jnp.float32),
             pltpu.VMEM((CH, ACC_W), jnp.float32)),
            pltpu.VMEM((NC * NS, H, L), jnp.float32),
            pltpu.VMEM_SHARED((NR, ACC_W), jnp.float32),
            (pltpu.SemaphoreType.DMA, pltpu.SemaphoreType.DMA),
            (pltpu.SemaphoreType.DMA, pltpu.SemaphoreType.DMA),
        ],
        compiler_params=_sc_params,
    )()
    return f(hp, tp, logitsT, tilemax, C0, C1)


def kernel(relation_triplets, rel_emb, proj1_W, proj1_b,
           l0_attn_W, l0_attn_b, l0_attn_bin, l0_attn_vec,
           l0_aggr_W, l0_aggr_b, l0_res_W, l0_res_b,
           l1_attn_W, l1_attn_b, l1_attn_bin, l1_attn_vec,
           l1_aggr_W, l1_aggr_b, l1_res_W, l1_res_b):
    E = relation_triplets.shape[0]
    tri = relation_triplets.astype(jnp.int32)
    pad = E_PAD - E
    hp = jnp.concatenate([tri[:, 0], jnp.zeros((pad,), jnp.int32)])
    tp = jnp.concatenate([tri[:, 1], jnp.zeros((pad,), jnp.int32)])
    bp = jnp.concatenate([tri[:, 2], jnp.zeros((pad,), jnp.int32)])

    embp = jnp.pad(rel_emb, ((0, NP - NR), (0, 0)))
    emb = _tc_proj(embp, proj1_W, proj1_b.reshape(1, DIM))

    layers = [
        (l0_attn_W, l0_attn_b, l0_attn_bin, l0_attn_vec,
         l0_aggr_W, l0_aggr_b, l0_res_W, l0_res_b),
        (l1_attn_W, l1_attn_b, l1_attn_bin, l1_attn_vec,
         l1_aggr_W, l1_aggr_b, l1_res_W, l1_res_b),
    ]
    out = None
    for (attn_W, attn_b, attn_bin, attn_vec,
         aggr_W, aggr_b, res_W, res_b) in layers:
        A, Bm, C0, C1, R = _tc_mm(
            emb, attn_W[:DIM], attn_W[DIM:], attn_b.reshape(1, DIM),
            aggr_W, aggr_b.reshape(1, DIM), res_W, res_b.reshape(1, DIM),
        )
        nbin = attn_bin.shape[0]
        binp = jnp.pad(attn_bin.reshape(nbin, H), ((0, 0), (0, L - H)))
        vecb = jnp.broadcast_to(attn_vec.reshape(DIM, 1), (DIM, L))
        logitsT, tilemax = _sc_logits(hp, tp, bp, A, Bm, binp, vecb)
        accum = _sc_aggregate(hp, tp, logitsT, tilemax, C0, C1)
        out = _tc_combine(
            accum[0, :, :128], accum[0, :, 128:132],
            accum[1, :, :128], accum[1, :, 128:132], R,
        )
        emb = jnp.pad(out, ((0, NP - NR), (0, 0)))
    return out


# speedup vs baseline: 14.5272x; 2.0582x over previous
import functools

import jax
import jax.numpy as jnp
from jax import lax
from jax.experimental import pallas as pl
from jax.experimental.pallas import tpu as pltpu
from jax.experimental.pallas import tpu_sc as plsc

NR = 10000
DIM = 256
H = 8
DH = 32
NP = 10240
L = 16
NC = 2
NS = 16
E_PAD = 163840
CH1 = 80
CH = 64
EPT1 = E_PAD // (NC * NS)
NCH1 = EPT1 // CH1
EPT2 = E_PAD // NS
NCH2 = EPT2 // CH
ACC_W = 144
E_REAL = 160000
RPT = NR // NS
WCH = 25

_mesh = plsc.VectorSubcoreMesh(
    core_axis_name="c", subcore_axis_name="s", num_cores=NC, num_subcores=NS
)
_sc_params = pltpu.CompilerParams(
    use_tc_tiling_on_sc=False, needs_layout_passes=False
)


def _leaky(x):
    return jnp.where(x >= 0, x, 0.2 * x)


def _proj_body(e_ref, w_ref, b_ref, o_ref):
    o_ref[...] = jax.nn.relu(
        jnp.dot(e_ref[...], w_ref[...], preferred_element_type=jnp.float32)
        + b_ref[...]
    )


def _tc_proj(embp, W, b):
    blk = 256
    return pl.pallas_call(
        _proj_body,
        grid=(NP // blk,),
        in_specs=[
            pl.BlockSpec((blk, DIM), lambda i: (i, 0)),
            pl.BlockSpec((DIM, DIM), lambda i: (0, 0)),
            pl.BlockSpec((1, DIM), lambda i: (0, 0)),
        ],
        out_specs=pl.BlockSpec((blk, DIM), lambda i: (i, 0)),
        out_shape=jax.ShapeDtypeStruct((NP, DIM), jnp.float32),
    )(embp, W, b)


def _mm_body(e_ref, wh_ref, wt_ref, ab_ref, aw_ref, agb_ref, rw_ref, rb_ref,
             a_ref, b_ref, c0_ref, c1_ref, r_ref):
    e = e_ref[...]
    a_ref[...] = jnp.dot(e, wh_ref[...], preferred_element_type=jnp.float32)
    b_ref[...] = (
        jnp.dot(e, wt_ref[...], preferred_element_type=jnp.float32) + ab_ref[...]
    )
    c = jnp.dot(e, aw_ref[...], preferred_element_type=jnp.float32) + agb_ref[...]
    c0_ref[...] = c[:, :128]
    c1_ref[...] = c[:, 128:]
    r_ref[...] = jax.nn.relu(
        jnp.dot(e, rw_ref[...], preferred_element_type=jnp.float32) + rb_ref[...]
    )


def _tc_mm(embp, Wh, Wt, attn_b, aggr_W, aggr_b, res_W, res_b):
    blk = 256
    wspec = pl.BlockSpec((DIM, DIM), lambda i: (0, 0))
    bspec = pl.BlockSpec((1, DIM), lambda i: (0, 0))
    ospec = pl.BlockSpec((blk, DIM), lambda i: (i, 0))
    hspec = pl.BlockSpec((blk, 128), lambda i: (i, 0))
    return pl.pallas_call(
        _mm_body,
        grid=(NP // blk,),
        in_specs=[
            pl.BlockSpec((blk, DIM), lambda i: (i, 0)),
            wspec, wspec, bspec, wspec, bspec, wspec, bspec,
        ],
        out_specs=[ospec, ospec, hspec, hspec, ospec],
        out_shape=[
            jax.ShapeDtypeStruct((NP, DIM), jnp.float32),
            jax.ShapeDtypeStruct((NP, DIM), jnp.float32),
            jax.ShapeDtypeStruct((NP, 128), jnp.float32),
            jax.ShapeDtypeStruct((NP, 128), jnp.float32),
            jax.ShapeDtypeStruct((NP, DIM), jnp.float32),
        ],
    )(embp, Wh, Wt, attn_b, aggr_W, aggr_b, res_W, res_b)


def _combine_body(m0_ref, s0_ref, m1_ref, s1_ref, r_ref, o_ref):
    eps = 1e-16
    outs = []
    for m_ref, s_ref in ((m0_ref, s0_ref), (m1_ref, s1_ref)):
        m = m_ref[...]
        s = s_ref[...]
        d = jnp.concatenate(
            [jnp.broadcast_to(s[:, k:k + 1], (m.shape[0], DH)) for k in range(4)],
            axis=1,
        )
        outs.append(m / (d + eps))
    o = jnp.concatenate(outs, axis=1)
    o_ref[...] = jax.nn.relu(o) + r_ref[...]


def _tc_combine(m0, s0, m1, s1, R):
    blk = 80
    mspec = pl.BlockSpec((blk, 128), lambda i: (i, 0))
    sspec = pl.BlockSpec((blk, 4), lambda i: (i, 0))
    return pl.pallas_call(
        _combine_body,
        grid=(NR // blk,),
        in_specs=[mspec, sspec, mspec, sspec,
                  pl.BlockSpec((blk, DIM), lambda i: (i, 0))],
        out_specs=pl.BlockSpec((blk, DIM), lambda i: (i, 0)),
        out_shape=jax.ShapeDtypeStruct((NR, DIM), jnp.float32),
    )(m0, s0, m1, s1, R)


def _sc_logits_body(hp, tp, bp, A, B, binp, vecb,
                    logitsT, tilemax,
                    hbufs, tbufs, bbufs, arowss, browss, binrowss, vecv,
                    lbuf, maxb, sems):
    cid = lax.axis_index("c")
    sid = lax.axis_index("s")
    wid = sid * NC + cid
    base = wid * EPT1
    iot = lax.iota(jnp.int32, L)
    pltpu.sync_copy(vecb, vecv)

    vregs_vec = [vecv[pl.ds(i * L, L)] for i in range(DIM // L)]

    def issue(off, p):
        pltpu.sync_copy(hp.at[pl.ds(off, CH1)], hbufs[p])
        pltpu.sync_copy(tp.at[pl.ds(off, CH1)], tbufs[p])
        pltpu.sync_copy(bp.at[pl.ds(off, CH1)], bbufs[p])
        pltpu.async_copy(A.at[hbufs[p]], arowss[p], sems[p])
        pltpu.async_copy(B.at[tbufs[p]], browss[p], sems[p])
        pltpu.async_copy(binp.at[bbufs[p]], binrowss[p], sems[p])

    def drain(p):
        pltpu.make_async_copy(A.at[hbufs[p]], arowss[p], sems[p]).wait()
        pltpu.make_async_copy(B.at[tbufs[p]], browss[p], sems[p]).wait()
        pltpu.make_async_copy(binp.at[bbufs[p]], binrowss[p], sems[p]).wait()

    ohs = [(iot == k).astype(jnp.float32) for k in range(H)]

    def compute(off, p, maxv):
        arows, brows, binrows = arowss[p], browss[p], binrowss[p]

        def edge_body(e, mv):
            row = _leaky(binrows[e, :])
            for k in range(H):
                acc = jnp.zeros((L,), jnp.float32)
                for j in range(DH // L):
                    col = k * DH + j * L
                    a = arows[e, pl.ds(col, L)]
                    b = brows[e, pl.ds(col, L)]
                    acc = acc + _leaky(a + b) * vregs_vec[k * 2 + j]
                row = row + jnp.sum(acc) * ohs[k]
            lbuf[e, :] = row
            return jnp.maximum(mv, row)

        maxv = pl.loop(0, CH1, init_carry=maxv)(edge_body)
        pltpu.sync_copy(lbuf, logitsT.at[pl.ds(off, CH1)])
        return maxv

    issue(base, 0)

    def pair_body(cj, maxv):
        for p in range(2):
            ci = 2 * cj + p

            @pl.when(ci + 1 < NCH1)
            def _():
                issue(base + (ci + 1) * CH1, 1 - p)

            drain(p)
            maxv = compute(base + ci * CH1, p, maxv)
        return maxv

    maxv = pl.loop(0, NCH1 // 2,
                   init_carry=jnp.full((L,), -1e30, jnp.float32))(pair_body)
    maxb[:] = maxv
    pltpu.sync_copy(maxb, tilemax.at[wid])


def _sc_logits(hp, tp, bp, A, B, binp, vecb):
    ibuf = pltpu.VMEM((CH1,), jnp.int32)
    rbuf = pltpu.VMEM((CH1, DIM), jnp.float32)
    nbuf = pltpu.VMEM((CH1, L), jnp.float32)
    f = functools.partial(
        pl.kernel,
        _sc_logits_body,
        out_type=[
            jax.ShapeDtypeStruct((E_PAD, L), jnp.float32),
            jax.ShapeDtypeStruct((NC * NS, L), jnp.float32),
        ],
        mesh=_mesh,
        scratch_types=[
            (ibuf, ibuf), (ibuf, ibuf), (ibuf, ibuf),
            (rbuf, rbuf), (rbuf, rbuf), (nbuf, nbuf),
            pltpu.VMEM((DIM,), jnp.float32),
            pltpu.VMEM((CH1, L), jnp.float32),
            pltpu.VMEM((L,), jnp.float32),
            (pltpu.SemaphoreType.DMA, pltpu.SemaphoreType.DMA),
        ],
        compiler_params=_sc_params,
    )()
    return f(hp, tp, bp, A, B, binp, vecb)


def _sc_aggr_body(hp, tp, logitsT, tilemax, C0, C1,
                  accum,
                  hbufs, tbufs, abufs, lbufs, crowss, msgs, tmaxv,
                  oshared, gsems, asems):
    cid = lax.axis_index("c")
    sid = lax.axis_index("s")
    iot = lax.iota(jnp.int32, L)
    pltpu.sync_copy(tilemax, tmaxv)

    def wbody(w, acc):
        return jnp.maximum(acc, tmaxv[w, :])
    gmax = pl.loop(0, NC * NS,
                   init_carry=jnp.full((L,), -1e30, jnp.float32))(wbody)
    Ks = [jnp.where(cid == 0, gmax[k], gmax[k + 4]) for k in range(4)]
    ohs = [(iot == k).astype(jnp.float32) for k in range(L)]
    kvec = jnp.zeros((L,), jnp.float32)
    for k in range(4):
        kvec = kvec + Ks[k] * (ohs[k] + ohs[k + 4])

    for p in range(2):
        @pl.loop(0, CH)
        def _(r, _p=p):
            for j in range(ACC_W // L):
                msgs[_p][r, pl.ds(j * L, L)] = jnp.zeros((L,), jnp.float32)

    for q in range(RPT // WCH):
        pltpu.sync_copy(
            msgs[0].at[pl.ds(0, WCH)],
            oshared.at[pl.ds(sid * RPT + q * WCH, WCH)],
        )
    plsc.subcore_barrier()

    base = sid * EPT2

    def issue(off, p):
        pltpu.sync_copy(hp.at[pl.ds(off, CH)], hbufs[p])
        pltpu.sync_copy(tp.at[pl.ds(off, CH)], tbufs[p])
        pltpu.sync_copy(logitsT.at[pl.ds(off, CH)], lbufs[p])

        @pl.when(cid == 0)
        def _():
            pltpu.async_copy(C0.at[tbufs[p]], crowss[p], gsems[p])

        @pl.when(cid == 1)
        def _():
            pltpu.async_copy(C1.at[tbufs[p]], crowss[p], gsems[p])

    def drain_gather(p):
        @pl.when(cid == 0)
        def _():
            pltpu.make_async_copy(C0.at[tbufs[p]], crowss[p], gsems[p]).wait()

        @pl.when(cid == 1)
        def _():
            pltpu.make_async_copy(C1.at[tbufs[p]], crowss[p], gsems[p]).wait()

    def drain_add(p):
        pltpu.make_async_copy(
            msgs[p], oshared.at[abufs[p]], asems[p]
        ).wait()

    issue(base, 0)

    def pair_body(cj):
        for p in range(2):
            ci = 2 * cj + p
            off = base + ci * CH

            @pl.when(ci + 1 < NCH2)
            def _():
                issue(base + (ci + 1) * CH, 1 - p)

            drain_gather(p)

            @pl.when(cj > 0)
            def _():
                drain_add(p)

            lbuf, crows, msg = lbufs[p], crowss[p], msgs[p]

            @pl.loop(0, CH)
            def edge_body(e):
                live = jnp.where(off + e < E_REAL, 1.0, 0.0)
                vrow = jnp.exp(lbuf[e, :] - kvec) * live
                svec = jnp.zeros((L,), jnp.float32)
                for k in range(4):
                    v = jnp.where(cid == 0, vrow[k], vrow[k + 4])
                    svec = svec + v * ohs[k]
                    vb = jnp.full((L,), v)
                    for j in range(DH // L):
                        col = k * DH + j * L
                        msg[e, pl.ds(col, L)] = crows[e, pl.ds(col, L)] * vb
                msg[e, pl.ds(128, L)] = svec

            for j in range(CH // L):
                abufs[p][pl.ds(j * L, L)] = hbufs[p][pl.ds(j * L, L)]
            pltpu.async_copy(msg, oshared.at[abufs[p]], asems[p], add=True)

    pl.loop(0, NCH2 // 2)(pair_body)
    drain_add(0)
    drain_add(1)

    plsc.subcore_barrier()
    for q in range(RPT // WCH):
        rows = sid * RPT + q * WCH
        pltpu.sync_copy(
            oshared.at[pl.ds(rows, WCH)], msgs[0].at[pl.ds(0, WCH)]
        )
        pltpu.sync_copy(
            msgs[0].at[pl.ds(0, WCH)], accum.at[cid, pl.ds(rows, WCH)]
        )


def _sc_aggregate(hp, tp, logitsT, tilemax, C0, C1):
    ibuf = pltpu.VMEM((CH,), jnp.int32)
    f = functools.partial(
        pl.kernel,
        _sc_aggr_body,
        out_type=jax.ShapeDtypeStruct((NC, NR, ACC_W), jnp.float32),
        mesh=_mesh,
        scratch_types=[
            (ibuf, ibuf), (ibuf, ibuf), (ibuf, ibuf),
            (pltpu.VMEM((CH, L), jnp.float32),
             pltpu.VMEM((CH, L), jnp.float32)),
            (pltpu.VMEM((CH, 128), jnp.float32),
             pltpu.VMEM((CH, 128), jnp.float32)),
            (pltpu.VMEM((CH, ACC_W), jnp.float32),
             pltpu.VMEM((CH, ACC_W), jnp.float32)),
            pltpu.VMEM((NC * NS, L), jnp.float32),
            pltpu.VMEM_SHARED((NR, ACC_W), jnp.float32),
            (pltpu.SemaphoreType.DMA, pltpu.SemaphoreType.DMA),
            (pltpu.SemaphoreType.DMA, pltpu.SemaphoreType.DMA),
        ],
        compiler_params=_sc_params,
    )()
    return f(hp, tp, logitsT, tilemax, C0, C1)


def kernel(relation_triplets, rel_emb, proj1_W, proj1_b,
           l0_attn_W, l0_attn_b, l0_attn_bin, l0_attn_vec,
           l0_aggr_W, l0_aggr_b, l0_res_W, l0_res_b,
           l1_attn_W, l1_attn_b, l1_attn_bin, l1_attn_vec,
           l1_aggr_W, l1_aggr_b, l1_res_W, l1_res_b):
    E = relation_triplets.shape[0]
    tri = relation_triplets.astype(jnp.int32)
    pad = E_PAD - E
    hp = jnp.concatenate([tri[:, 0], jnp.zeros((pad,), jnp.int32)])
    tp = jnp.concatenate([tri[:, 1], jnp.zeros((pad,), jnp.int32)])
    bp = jnp.concatenate([tri[:, 2], jnp.zeros((pad,), jnp.int32)])

    embp = jnp.pad(rel_emb, ((0, NP - NR), (0, 0)))
    emb = _tc_proj(embp, proj1_W, proj1_b.reshape(1, DIM))

    layers = [
        (l0_attn_W, l0_attn_b, l0_attn_bin, l0_attn_vec,
         l0_aggr_W, l0_aggr_b, l0_res_W, l0_res_b),
        (l1_attn_W, l1_attn_b, l1_attn_bin, l1_attn_vec,
         l1_aggr_W, l1_aggr_b, l1_res_W, l1_res_b),
    ]
    out = None
    for (attn_W, attn_b, attn_bin, attn_vec,
         aggr_W, aggr_b, res_W, res_b) in layers:
        A, Bm, C0, C1, R = _tc_mm(
            emb, attn_W[:DIM], attn_W[DIM:], attn_b.reshape(1, DIM),
            aggr_W, aggr_b.reshape(1, DIM), res_W, res_b.reshape(1, DIM),
        )
        nbin = attn_bin.shape[0]
        binp = jnp.pad(attn_bin.reshape(nbin, H), ((0, 0), (0, L - H)))
        vecb = attn_vec.reshape(DIM)
        logitsT, tilemax = _sc_logits(hp, tp, bp, A, Bm, binp, vecb)
        accum = _sc_aggregate(hp, tp, logitsT, tilemax, C0, C1)
        out = _tc_combine(
            accum[0, :, :128], accum[0, :, 128:132],
            accum[1, :, :128], accum[1, :, 128:132], R,
        )
        emb = jnp.pad(out, ((0, NP - NR), (0, 0)))
    return out


# speedup vs baseline: 14.9693x; 1.0304x over previous
import functools

import jax
import jax.numpy as jnp
from jax import lax
from jax.experimental import pallas as pl
from jax.experimental.pallas import tpu as pltpu
from jax.experimental.pallas import tpu_sc as plsc

NR = 10000
DIM = 256
H = 8
DH = 32
NP = 10240
L = 16
NC = 2
NS = 16
E_PAD = 163840
CH1 = 80
CH = 64
EPT1 = E_PAD // (NC * NS)
NCH1 = EPT1 // CH1
EPT2 = E_PAD // NS
NCH2 = EPT2 // CH
ACC_W = 144
E_REAL = 160000
RPT = NR // NS
WCH = 25

_mesh = plsc.VectorSubcoreMesh(
    core_axis_name="c", subcore_axis_name="s", num_cores=NC, num_subcores=NS
)
_sc_params = pltpu.CompilerParams(
    use_tc_tiling_on_sc=False, needs_layout_passes=False
)


def _leaky(x):
    return jnp.where(x >= 0, x, 0.2 * x)


def _proj_body(e_ref, w_ref, b_ref, o_ref):
    o_ref[...] = jax.nn.relu(
        jnp.dot(e_ref[...], w_ref[...], preferred_element_type=jnp.float32)
        + b_ref[...]
    )


def _tc_proj(embp, W, b):
    blk = 256
    return pl.pallas_call(
        _proj_body,
        grid=(NP // blk,),
        in_specs=[
            pl.BlockSpec((blk, DIM), lambda i: (i, 0)),
            pl.BlockSpec((DIM, DIM), lambda i: (0, 0)),
            pl.BlockSpec((1, DIM), lambda i: (0, 0)),
        ],
        out_specs=pl.BlockSpec((blk, DIM), lambda i: (i, 0)),
        out_shape=jax.ShapeDtypeStruct((NP, DIM), jnp.float32),
    )(embp, W, b)


def _proj_mm_body(e_ref, pw_ref, pb_ref,
                  wh_ref, wt_ref, ab_ref, aw_ref, agb_ref, rw_ref, rb_ref,
                  a_ref, b_ref, c0_ref, c1_ref, r_ref):
    e = jax.nn.relu(
        jnp.dot(e_ref[...], pw_ref[...], preferred_element_type=jnp.float32)
        + pb_ref[...]
    )
    _emit_mm(e, wh_ref, wt_ref, ab_ref, aw_ref, agb_ref, rw_ref, rb_ref,
             a_ref, b_ref, c0_ref, c1_ref, r_ref)


def _comb_mm_body(m0_ref, s0_ref, m1_ref, s1_ref, r_ref,
                  wh_ref, wt_ref, ab_ref, aw_ref, agb_ref, rw_ref, rb_ref,
                  a_ref, b_ref, c0_ref, c1_ref, rn_ref):
    e = _combine_val(m0_ref, s0_ref, m1_ref, s1_ref, r_ref)
    _emit_mm(e, wh_ref, wt_ref, ab_ref, aw_ref, agb_ref, rw_ref, rb_ref,
             a_ref, b_ref, c0_ref, c1_ref, rn_ref)


def _emit_mm(e, wh_ref, wt_ref, ab_ref, aw_ref, agb_ref, rw_ref, rb_ref,
             a_ref, b_ref, c0_ref, c1_ref, r_ref):
    a_ref[...] = jnp.dot(e, wh_ref[...], preferred_element_type=jnp.float32)
    b_ref[...] = (
        jnp.dot(e, wt_ref[...], preferred_element_type=jnp.float32) + ab_ref[...]
    )
    c = jnp.dot(e, aw_ref[...], preferred_element_type=jnp.float32) + agb_ref[...]
    c0_ref[...] = c[:, :128]
    c1_ref[...] = c[:, 128:]
    r_ref[...] = jax.nn.relu(
        jnp.dot(e, rw_ref[...], preferred_element_type=jnp.float32) + rb_ref[...]
    )


def _mm_specs(nrows, blk):
    wspec = pl.BlockSpec((DIM, DIM), lambda i: (0, 0))
    bspec = pl.BlockSpec((1, DIM), lambda i: (0, 0))
    ospec = pl.BlockSpec((blk, DIM), lambda i: (i, 0))
    hspec = pl.BlockSpec((blk, 128), lambda i: (i, 0))
    w_in = [wspec, wspec, bspec, wspec, bspec, wspec, bspec]
    outs = [ospec, ospec, hspec, hspec, ospec]
    shapes = [
        jax.ShapeDtypeStruct((nrows, DIM), jnp.float32),
        jax.ShapeDtypeStruct((nrows, DIM), jnp.float32),
        jax.ShapeDtypeStruct((nrows, 128), jnp.float32),
        jax.ShapeDtypeStruct((nrows, 128), jnp.float32),
        jax.ShapeDtypeStruct((nrows, DIM), jnp.float32),
    ]
    return w_in, outs, shapes


def _tc_proj_mm(embp, pW, pb, Wh, Wt, attn_b, aggr_W, aggr_b, res_W, res_b):
    blk = 256
    w_in, outs, shapes = _mm_specs(NP, blk)
    return pl.pallas_call(
        _proj_mm_body,
        grid=(NP // blk,),
        in_specs=[
            pl.BlockSpec((blk, DIM), lambda i: (i, 0)),
            pl.BlockSpec((DIM, DIM), lambda i: (0, 0)),
            pl.BlockSpec((1, DIM), lambda i: (0, 0)),
        ] + w_in,
        out_specs=outs,
        out_shape=shapes,
    )(embp, pW, pb, Wh, Wt, attn_b, aggr_W, aggr_b, res_W, res_b)


def _tc_comb_mm(m0, s0, m1, s1, R,
                Wh, Wt, attn_b, aggr_W, aggr_b, res_W, res_b):
    blk = 80
    w_in, outs, shapes = _mm_specs(NR, blk)
    mspec = pl.BlockSpec((blk, 128), lambda i: (i, 0))
    sspec = pl.BlockSpec((blk, 4), lambda i: (i, 0))
    return pl.pallas_call(
        _comb_mm_body,
        grid=(NR // blk,),
        in_specs=[mspec, sspec, mspec, sspec,
                  pl.BlockSpec((blk, DIM), lambda i: (i, 0))] + w_in,
        out_specs=outs,
        out_shape=shapes,
    )(m0, s0, m1, s1, R, Wh, Wt, attn_b, aggr_W, aggr_b, res_W, res_b)


def _combine_val(m0_ref, s0_ref, m1_ref, s1_ref, r_ref):
    eps = 1e-16
    outs = []
    for m_ref, s_ref in ((m0_ref, s0_ref), (m1_ref, s1_ref)):
        m = m_ref[...]
        s = s_ref[...]
        d = jnp.concatenate(
            [jnp.broadcast_to(s[:, k:k + 1], (m.shape[0], DH)) for k in range(4)],
            axis=1,
        )
        outs.append(m / (d + eps))
    o = jnp.concatenate(outs, axis=1)
    return jax.nn.relu(o) + r_ref[...]


def _combine_body(m0_ref, s0_ref, m1_ref, s1_ref, r_ref, o_ref):
    o_ref[...] = _combine_val(m0_ref, s0_ref, m1_ref, s1_ref, r_ref)


def _tc_combine(m0, s0, m1, s1, R):
    blk = 80
    mspec = pl.BlockSpec((blk, 128), lambda i: (i, 0))
    sspec = pl.BlockSpec((blk, 4), lambda i: (i, 0))
    return pl.pallas_call(
        _combine_body,
        grid=(NR // blk,),
        in_specs=[mspec, sspec, mspec, sspec,
                  pl.BlockSpec((blk, DIM), lambda i: (i, 0))],
        out_specs=pl.BlockSpec((blk, DIM), lambda i: (i, 0)),
        out_shape=jax.ShapeDtypeStruct((NR, DIM), jnp.float32),
    )(m0, s0, m1, s1, R)


def _sc_logits_body(hp, tp, bp, A, B, binp, vecb,
                    logitsT, tilemax,
                    hbufs, tbufs, bbufs, arowss, browss, binrowss, vecv,
                    lbuf, maxb, sems):
    cid = lax.axis_index("c")
    sid = lax.axis_index("s")
    wid = sid * NC + cid
    base = wid * EPT1
    iot = lax.iota(jnp.int32, L)
    pltpu.sync_copy(vecb, vecv)

    vregs_vec = [vecv[pl.ds(i * L, L)] for i in range(DIM // L)]

    def issue(off, p):
        pltpu.sync_copy(hp.at[pl.ds(off, CH1)], hbufs[p])
        pltpu.sync_copy(tp.at[pl.ds(off, CH1)], tbufs[p])
        pltpu.sync_copy(bp.at[pl.ds(off, CH1)], bbufs[p])
        pltpu.async_copy(A.at[hbufs[p]], arowss[p], sems[p])
        pltpu.async_copy(B.at[tbufs[p]], browss[p], sems[p])
        pltpu.async_copy(binp.at[bbufs[p]], binrowss[p], sems[p])

    def drain(p):
        pltpu.make_async_copy(A.at[hbufs[p]], arowss[p], sems[p]).wait()
        pltpu.make_async_copy(B.at[tbufs[p]], browss[p], sems[p]).wait()
        pltpu.make_async_copy(binp.at[bbufs[p]], binrowss[p], sems[p]).wait()

    ohs = [(iot == k).astype(jnp.float32) for k in range(H)]

    def compute(off, p, maxv):
        arows, brows, binrows = arowss[p], browss[p], binrowss[p]

        def edge_body(e, mv):
            row = _leaky(binrows[e, :])
            for k in range(H):
                acc = jnp.zeros((L,), jnp.float32)
                for j in range(DH // L):
                    col = k * DH + j * L
                    a = arows[e, pl.ds(col, L)]
                    b = brows[e, pl.ds(col, L)]
                    acc = acc + _leaky(a + b) * vregs_vec[k * 2 + j]
                row = row + jnp.sum(acc) * ohs[k]
            lbuf[e, :] = row
            return jnp.maximum(mv, row)

        maxv = pl.loop(0, CH1, init_carry=maxv)(edge_body)
        pltpu.sync_copy(lbuf, logitsT.at[pl.ds(off, CH1)])
        return maxv

    issue(base, 0)

    def pair_body(cj, maxv):
        for p in range(2):
            ci = 2 * cj + p

            @pl.when(ci + 1 < NCH1)
            def _():
                issue(base + (ci + 1) * CH1, 1 - p)

            drain(p)
            maxv = compute(base + ci * CH1, p, maxv)
        return maxv

    maxv = pl.loop(0, NCH1 // 2,
                   init_carry=jnp.full((L,), -1e30, jnp.float32))(pair_body)
    maxb[:] = maxv
    pltpu.sync_copy(maxb, tilemax.at[wid])


def _sc_logits(hp, tp, bp, A, B, binp, vecb):
    ibuf = pltpu.VMEM((CH1,), jnp.int32)
    rbuf = pltpu.VMEM((CH1, DIM), jnp.float32)
    nbuf = pltpu.VMEM((CH1, L), jnp.float32)
    f = functools.partial(
        pl.kernel,
        _sc_logits_body,
        out_type=[
            jax.ShapeDtypeStruct((E_PAD, L), jnp.float32),
            jax.ShapeDtypeStruct((NC * NS, L), jnp.float32),
        ],
        mesh=_mesh,
        scratch_types=[
            (ibuf, ibuf), (ibuf, ibuf), (ibuf, ibuf),
            (rbuf, rbuf), (rbuf, rbuf), (nbuf, nbuf),
            pltpu.VMEM((DIM,), jnp.float32),
            pltpu.VMEM((CH1, L), jnp.float32),
            pltpu.VMEM((L,), jnp.float32),
            (pltpu.SemaphoreType.DMA, pltpu.SemaphoreType.DMA),
        ],
        compiler_params=_sc_params,
    )()
    return f(hp, tp, bp, A, B, binp, vecb)


def _sc_aggr_body(hp, tp, logitsT, tilemax, C0, C1,
                  accum,
                  hbufs, tbufs, abufs, lbufs, crowss, msgs, tmaxv,
                  oshared, gsems, asems):
    cid = lax.axis_index("c")
    sid = lax.axis_index("s")
    iot = lax.iota(jnp.int32, L)
    pltpu.sync_copy(tilemax, tmaxv)

    def wbody(w, acc):
        return jnp.maximum(acc, tmaxv[w, :])
    gmax = pl.loop(0, NC * NS,
                   init_carry=jnp.full((L,), -1e30, jnp.float32))(wbody)
    Ks = [jnp.where(cid == 0, gmax[k], gmax[k + 4]) for k in range(4)]
    ohs = [(iot == k).astype(jnp.float32) for k in range(L)]
    kvec = jnp.zeros((L,), jnp.float32)
    for k in range(4):
        kvec = kvec + Ks[k] * (ohs[k] + ohs[k + 4])

    for p in range(2):
        @pl.loop(0, CH)
        def _(r, _p=p):
            for j in range(ACC_W // L):
                msgs[_p][r, pl.ds(j * L, L)] = jnp.zeros((L,), jnp.float32)

    for q in range(RPT // WCH):
        pltpu.sync_copy(
            msgs[0].at[pl.ds(0, WCH)],
            oshared.at[pl.ds(sid * RPT + q * WCH, WCH)],
        )
    plsc.subcore_barrier()

    base = sid * EPT2

    def issue(off, p):
        pltpu.sync_copy(hp.at[pl.ds(off, CH)], hbufs[p])
        pltpu.sync_copy(tp.at[pl.ds(off, CH)], tbufs[p])
        pltpu.sync_copy(logitsT.at[pl.ds(off, CH)], lbufs[p])

        @pl.when(cid == 0)
        def _():
            pltpu.async_copy(C0.at[tbufs[p]], crowss[p], gsems[p])

        @pl.when(cid == 1)
        def _():
            pltpu.async_copy(C1.at[tbufs[p]], crowss[p], gsems[p])

    def drain_gather(p):
        @pl.when(cid == 0)
        def _():
            pltpu.make_async_copy(C0.at[tbufs[p]], crowss[p], gsems[p]).wait()

        @pl.when(cid == 1)
        def _():
            pltpu.make_async_copy(C1.at[tbufs[p]], crowss[p], gsems[p]).wait()

    def drain_add(p):
        pltpu.make_async_copy(
            msgs[p], oshared.at[abufs[p]], asems[p]
        ).wait()

    issue(base, 0)

    def pair_body(cj):
        for p in range(2):
            ci = 2 * cj + p
            off = base + ci * CH

            @pl.when(ci + 1 < NCH2)
            def _():
                issue(base + (ci + 1) * CH, 1 - p)

            drain_gather(p)

            @pl.when(cj > 0)
            def _():
                drain_add(p)

            lbuf, crows, msg = lbufs[p], crowss[p], msgs[p]

            @pl.loop(0, CH)
            def edge_body(e):
                live = jnp.where(off + e < E_REAL, 1.0, 0.0)
                vrow = jnp.exp(lbuf[e, :] - kvec) * live
                svec = jnp.zeros((L,), jnp.float32)
                for k in range(4):
                    v = jnp.where(cid == 0, vrow[k], vrow[k + 4])
                    svec = svec + v * ohs[k]
                    vb = jnp.full((L,), v)
                    for j in range(DH // L):
                        col = k * DH + j * L
                        msg[e, pl.ds(col, L)] = crows[e, pl.ds(col, L)] * vb
                msg[e, pl.ds(128, L)] = svec

            for j in range(CH // L):
                abufs[p][pl.ds(j * L, L)] = hbufs[p][pl.ds(j * L, L)]
            pltpu.async_copy(msg, oshared.at[abufs[p]], asems[p], add=True)

    pl.loop(0, NCH2 // 2)(pair_body)
    drain_add(0)
    drain_add(1)

    plsc.subcore_barrier()
    for q in range(RPT // WCH):
        rows = sid * RPT + q * WCH
        pltpu.sync_copy(
            oshared.at[pl.ds(rows, WCH)], msgs[0].at[pl.ds(0, WCH)]
        )
        pltpu.sync_copy(
            msgs[0].at[pl.ds(0, WCH)], accum.at[cid, pl.ds(rows, WCH)]
        )


def _sc_aggregate(hp, tp, logitsT, tilemax, C0, C1):
    ibuf = pltpu.VMEM((CH,), jnp.int32)
    f = functools.partial(
        pl.kernel,
        _sc_aggr_body,
        out_type=jax.ShapeDtypeStruct((NC, NR, ACC_W), jnp.float32),
        mesh=_mesh,
        scratch_types=[
            (ibuf, ibuf), (ibuf, ibuf), (ibuf, ibuf),
            (pltpu.VMEM((CH, L), jnp.float32),
             pltpu.VMEM((CH, L), jnp.float32)),
            (pltpu.VMEM((CH, 128), jnp.float32),
             pltpu.VMEM((CH, 128), jnp.float32)),
            (pltpu.VMEM((CH, ACC_W), jnp.float32),
             pltpu.VMEM((CH, ACC_W), jnp.float32)),
            pltpu.VMEM((NC * NS, L), jnp.float32),
            pltpu.VMEM_SHARED((NR, ACC_W), jnp.float32),
            (pltpu.SemaphoreType.DMA, pltpu.SemaphoreType.DMA),
            (pltpu.SemaphoreType.DMA, pltpu.SemaphoreType.DMA),
        ],
        compiler_params=_sc_params,
    )()
    return f(hp, tp, logitsT, tilemax, C0, C1)


def kernel(relation_triplets, rel_emb, proj1_W, proj1_b,
           l0_attn_W, l0_attn_b, l0_attn_bin, l0_attn_vec,
           l0_aggr_W, l0_aggr_b, l0_res_W, l0_res_b,
           l1_attn_W, l1_attn_b, l1_attn_bin, l1_attn_vec,
           l1_aggr_W, l1_aggr_b, l1_res_W, l1_res_b):
    E = relation_triplets.shape[0]
    tri = relation_triplets.astype(jnp.int32)
    pad = E_PAD - E
    hp = jnp.concatenate([tri[:, 0], jnp.zeros((pad,), jnp.int32)])
    tp = jnp.concatenate([tri[:, 1], jnp.zeros((pad,), jnp.int32)])
    bp = jnp.concatenate([tri[:, 2], jnp.zeros((pad,), jnp.int32)])

    embp = jnp.pad(rel_emb, ((0, NP - NR), (0, 0)))

    def sc_layer(A, Bm, C0, C1, attn_bin, attn_vec):
        nbin = attn_bin.shape[0]
        binp = jnp.pad(attn_bin.reshape(nbin, H), ((0, 0), (0, L - H)))
        vecb = attn_vec.reshape(DIM)
        logitsT, tilemax = _sc_logits(hp, tp, bp, A, Bm, binp, vecb)
        return _sc_aggregate(hp, tp, logitsT, tilemax, C0, C1)

    A, Bm, C0, C1, R0 = _tc_proj_mm(
        embp, proj1_W, proj1_b.reshape(1, DIM),
        l0_attn_W[:DIM], l0_attn_W[DIM:], l0_attn_b.reshape(1, DIM),
        l0_aggr_W, l0_aggr_b.reshape(1, DIM), l0_res_W,
        l0_res_b.reshape(1, DIM),
    )
    acc0 = sc_layer(A, Bm, C0, C1, l0_attn_bin, l0_attn_vec)
    A, Bm, C0, C1, R1 = _tc_comb_mm(
        acc0[0, :, :128], acc0[0, :, 128:132],
        acc0[1, :, :128], acc0[1, :, 128:132], R0,
        l1_attn_W[:DIM], l1_attn_W[DIM:], l1_attn_b.reshape(1, DIM),
        l1_aggr_W, l1_aggr_b.reshape(1, DIM), l1_res_W,
        l1_res_b.reshape(1, DIM),
    )
    acc1 = sc_layer(A, Bm, C0, C1, l1_attn_bin, l1_attn_vec)
    return _tc_combine(
        acc1[0, :, :128], acc1[0, :, 128:132],
        acc1[1, :, :128], acc1[1, :, 128:132], R1,
    )
